# baseline (device time: 46798 ns/iter reference)
import jax
import jax.numpy as jnp
from jax import lax
from jax.experimental import pallas as pl
from jax.experimental.pallas import tpu as pltpu

N_DEV = 4
STEPS = N_DEV - 1
DIRS = (1, -1, 1, -1)
B = len(DIRS)
WIRE_DTYPE = jnp.bfloat16


def _coords(q):
    return (q // 2, (q % 2) ^ (q // 2))


def kernel(x):
    m, n = x.shape
    bandm = m // B
    segm = bandm // N_DEV

    def body(x_ref, out_ref, xv, acc, rs_sbuf, rs_tmp, ag_own, ag_buf,
             rs_send, rs_recv, ag_send, ag_recv, xin_sems, wb_sems):
        mx = lax.axis_index("x")
        my = lax.axis_index("y")
        p = 2 * mx + (my ^ mx)

        def seg_off(b, s):
            return b * bandm + s * segm

        def stage_in(b):
            return pltpu.make_async_copy(
                x_ref.at[pl.ds(b * bandm, bandm)],
                xv.at[pl.ds(b * bandm, bandm)],
                xin_sems.at[b],
            )

        def writeback(b):
            return pltpu.make_async_copy(
                acc.at[pl.ds(b * bandm, bandm)],
                out_ref.at[pl.ds(b * bandm, bandm)],
                wb_sems.at[b],
            )

        for b in range(B):
            stage_in(b).start()

        barrier_sem = pltpu.get_barrier_semaphore()
        for dq in (1, 3):
            pl.semaphore_signal(
                barrier_sem, inc=1,
                device_id=_coords(jnp.mod(p + dq, N_DEV)),
                device_id_type=pl.DeviceIdType.MESH,
            )
        pl.semaphore_wait(barrier_sem, 2)

        def rs_rdma(b, t):
            return pltpu.make_async_remote_copy(
                src_ref=rs_sbuf.at[b * STEPS + t],
                dst_ref=rs_tmp.at[b * STEPS + t],
                send_sem=rs_send.at[b * STEPS + t],
                recv_sem=rs_recv.at[b * STEPS + t],
                device_id=_coords(jnp.mod(p + DIRS[b], N_DEV)),
                device_id_type=pl.DeviceIdType.MESH,
            )

        def rs_start(b, t):
            d = DIRS[b]
            s = jnp.mod(p - d * t, N_DEV)
            src = xv if t == 0 else acc
            rs_sbuf[b * STEPS + t, :, :] = (
                src[pl.ds(seg_off(b, s), segm), :].astype(WIRE_DTYPE))
            rs_rdma(b, t).start()

        def ag_rdma(b, t):
            src = ag_own.at[b] if t == 0 else ag_buf.at[b * STEPS + t - 1]
            return pltpu.make_async_remote_copy(
                src_ref=src,
                dst_ref=ag_buf.at[b * STEPS + t],
                send_sem=ag_send.at[b * STEPS + t],
                recv_sem=ag_recv.at[b * STEPS + t],
                device_id=_coords(jnp.mod(p + DIRS[b], N_DEV)),
                device_id_type=pl.DeviceIdType.MESH,
            )

        for b in range(B):
            stage_in(b).wait()
            rs_start(b, 0)
        for t in range(STEPS):
            for b in range(B):
                d = DIRS[b]
                rs_rdma(b, t).wait_recv()
                s = jnp.mod(p - d * t - d, N_DEV)
                off = seg_off(b, s)
                acc[pl.ds(off, segm), :] = (
                    xv[pl.ds(off, segm), :]
                    + rs_tmp[b * STEPS + t].astype(jnp.float32))
                if t < STEPS - 1:
                    rs_start(b, t + 1)
                else:
                    ag_own[b, :, :] = acc[
                        pl.ds(seg_off(b, jnp.mod(p + d, N_DEV)), segm), :
                    ].astype(WIRE_DTYPE)
                    ag_rdma(b, 0).start()

        for t in range(STEPS):
            for b in range(B):
                d = DIRS[b]
                ag_rdma(b, t).wait_recv()
                if t < STEPS - 1:
                    ag_rdma(b, t + 1).start()
                off = seg_off(b, jnp.mod(p - d * t, N_DEV))
                acc[pl.ds(off, segm), :] = (
                    ag_buf[b * STEPS + t].astype(jnp.float32))
                if t == STEPS - 1:
                    writeback(b).start()

        for b in range(B):
            for t in range(STEPS):
                rs_rdma(b, t).wait_send()
                ag_rdma(b, t).wait_send()
            writeback(b).wait()

    return pl.pallas_call(
        body,
        out_shape=jax.ShapeDtypeStruct((m, n), jnp.float32),
        in_specs=[pl.BlockSpec(memory_space=pl.ANY)],
        out_specs=pl.BlockSpec(memory_space=pl.ANY),
        scratch_shapes=[
            pltpu.VMEM((m, n), jnp.float32),
            pltpu.VMEM((m, n), jnp.float32),
            pltpu.VMEM((B * STEPS, segm, n), WIRE_DTYPE),
            pltpu.VMEM((B * STEPS, segm, n), WIRE_DTYPE),
            pltpu.VMEM((B, segm, n), WIRE_DTYPE),
            pltpu.VMEM((B * STEPS, segm, n), WIRE_DTYPE),
            pltpu.SemaphoreType.DMA((B * STEPS,)),
            pltpu.SemaphoreType.DMA((B * STEPS,)),
            pltpu.SemaphoreType.DMA((B * STEPS,)),
            pltpu.SemaphoreType.DMA((B * STEPS,)),
            pltpu.SemaphoreType.DMA((B,)),
            pltpu.SemaphoreType.DMA((B,)),
        ],
        compiler_params=pltpu.CompilerParams(collective_id=0),
    )(x)


# device time: 44685 ns/iter; 1.0473x vs baseline; 1.0473x over previous
import jax
import jax.numpy as jnp
from jax import lax
from jax.experimental import pallas as pl
from jax.experimental.pallas import tpu as pltpu

N_DEV = 4
STEPS = N_DEV - 1
DIRS = (1, -1, 1, -1)
B = len(DIRS)
WIRE_DTYPE = jnp.bfloat16


def _coords(q):
    return (q // 2, (q % 2) ^ (q // 2))


def kernel(x):
    m, n = x.shape
    bandm = m // B
    segm = bandm // N_DEV

    def body(x_ref, out_ref, rs_sbuf, rs_tmp, ag_own, ag_buf,
             rs_send, rs_recv, ag_send, ag_recv):
        mx = lax.axis_index("x")
        my = lax.axis_index("y")
        p = 2 * mx + (my ^ mx)

        def seg_off(b, s):
            return b * bandm + s * segm

        def seg(ref, b, s):
            return ref.at[pl.ds(seg_off(b, s), segm)]

        barrier_sem = pltpu.get_barrier_semaphore()
        for dq in (1, 3):
            pl.semaphore_signal(
                barrier_sem, inc=1,
                device_id=_coords(jnp.mod(p + dq, N_DEV)),
                device_id_type=pl.DeviceIdType.MESH,
            )
        pl.semaphore_wait(barrier_sem, 2)

        def rs_rdma(b, t):
            return pltpu.make_async_remote_copy(
                src_ref=rs_sbuf.at[b * STEPS + t],
                dst_ref=rs_tmp.at[b * STEPS + t],
                send_sem=rs_send.at[b * STEPS + t],
                recv_sem=rs_recv.at[b * STEPS + t],
                device_id=_coords(jnp.mod(p + DIRS[b], N_DEV)),
                device_id_type=pl.DeviceIdType.MESH,
            )

        def rs_start(b, t):
            d = DIRS[b]
            s = jnp.mod(p - d * t, N_DEV)
            src = x_ref if t == 0 else out_ref
            rs_sbuf[b * STEPS + t, :, :] = (
                src[pl.ds(seg_off(b, s), segm), :].astype(WIRE_DTYPE))
            rs_rdma(b, t).start()

        def ag_rdma(b, t):
            src = ag_own.at[b] if t == 0 else ag_buf.at[b * STEPS + t - 1]
            return pltpu.make_async_remote_copy(
                src_ref=src,
                dst_ref=ag_buf.at[b * STEPS + t],
                send_sem=ag_send.at[b * STEPS + t],
                recv_sem=ag_recv.at[b * STEPS + t],
                device_id=_coords(jnp.mod(p + DIRS[b], N_DEV)),
                device_id_type=pl.DeviceIdType.MESH,
            )

        for b in range(B):
            rs_start(b, 0)
        for t in range(STEPS):
            for b in range(B):
                d = DIRS[b]
                rs_rdma(b, t).wait_recv()
                s = jnp.mod(p - d * t - d, N_DEV)
                off = seg_off(b, s)
                out_ref[pl.ds(off, segm), :] = (
                    x_ref[pl.ds(off, segm), :]
                    + rs_tmp[b * STEPS + t].astype(jnp.float32))
                if t < STEPS - 1:
                    rs_start(b, t + 1)
                else:
                    ag_own[b, :, :] = out_ref[
                        pl.ds(seg_off(b, jnp.mod(p + d, N_DEV)), segm), :
                    ].astype(WIRE_DTYPE)
                    ag_rdma(b, 0).start()

        for t in range(STEPS):
            for b in range(B):
                d = DIRS[b]
                ag_rdma(b, t).wait_recv()
                if t < STEPS - 1:
                    ag_rdma(b, t + 1).start()
                off = seg_off(b, jnp.mod(p - d * t, N_DEV))
                out_ref[pl.ds(off, segm), :] = (
                    ag_buf[b * STEPS + t].astype(jnp.float32))

        for b in range(B):
            for t in range(STEPS):
                rs_rdma(b, t).wait_send()
                ag_rdma(b, t).wait_send()

    return pl.pallas_call(
        body,
        out_shape=jax.ShapeDtypeStruct((m, n), jnp.float32),
        in_specs=[pl.BlockSpec(memory_space=pltpu.VMEM)],
        out_specs=pl.BlockSpec(memory_space=pltpu.VMEM),
        scratch_shapes=[
            pltpu.VMEM((B * STEPS, segm, n), WIRE_DTYPE),
            pltpu.VMEM((B * STEPS, segm, n), WIRE_DTYPE),
            pltpu.VMEM((B, segm, n), WIRE_DTYPE),
            pltpu.VMEM((B * STEPS, segm, n), WIRE_DTYPE),
            pltpu.SemaphoreType.DMA((B * STEPS,)),
            pltpu.SemaphoreType.DMA((B * STEPS,)),
            pltpu.SemaphoreType.DMA((B * STEPS,)),
            pltpu.SemaphoreType.DMA((B * STEPS,)),
        ],
        compiler_params=pltpu.CompilerParams(collective_id=0),
    )(x)
